# baseline (device time: 31071 ns/iter reference)
import jax
import jax.numpy as jnp
from jax import lax
from jax.experimental import pallas as pl
from jax.experimental.pallas import tpu as pltpu

N_DEV = 4
S = 2
F32 = jnp.float32
BF16 = jnp.bfloat16


def kernel(x):
    _, m, n = x.shape
    xs = x.reshape(m, n)
    chunk = n // N_DEV
    half = chunk // 2
    rows = m // S

    pp = lax.axis_index("i")
    h0r = lax.dynamic_slice(
        xs, (0, lax.rem(pp + N_DEV - 1, N_DEV) * chunk), (m, half))
    h0l = lax.dynamic_slice(
        xs, (0, lax.rem(pp + 1, N_DEV) * chunk + half), (m, half))
    xb0 = jnp.stack([h0r, h0l]).astype(BF16)

    def body(x_hbm, xb0_ref, out_ref, x_vmem, stage_sems,
             comm_r, comm_l, send_r, recv_r, send_l, recv_l):
        p = lax.axis_index("i")
        left = lax.rem(p + N_DEV - 1, N_DEV)
        right = lax.rem(p + 1, N_DEV)

        def col_r(h):
            return lax.rem(p + 2 * N_DEV - 2 - h, N_DEV) * chunk

        def col_l(h):
            return lax.rem(p + 2 + h, N_DEV) * chunk + half

        stage_copies = []
        for i, c in enumerate([col_r(0), col_r(1), col_l(1) - half,
                               col_r(2)]):
            cp = pltpu.make_async_copy(
                x_hbm.at[:, pl.ds(c, chunk)],
                x_vmem.at[:, pl.ds(c, chunk)],
                stage_sems.at[i])
            cp.start()
            stage_copies.append(cp)

        barrier_sem = pltpu.get_barrier_semaphore()
        for nbr in [left, right]:
            pl.semaphore_signal(
                barrier_sem, inc=1,
                device_id=(nbr,), device_id_type=pl.DeviceIdType.MESH,
            )

        pl.semaphore_wait(barrier_sem, 2)

        def make(h, s, direction):
            if direction == 0:
                src0 = (xb0_ref.at[0, pl.ds(s * rows, rows)]
                        if h == 0 else comm_r.at[h, s])
                return pltpu.make_async_remote_copy(
                    src_ref=src0,
                    dst_ref=comm_r.at[h + 1, s],
                    send_sem=send_r.at[h, s],
                    recv_sem=recv_r.at[h, s],
                    device_id=(right,),
                    device_id_type=pl.DeviceIdType.MESH,
                )
            else:
                src1 = (xb0_ref.at[1, pl.ds(s * rows, rows)]
                        if h == 0 else comm_l.at[h, s])
                return pltpu.make_async_remote_copy(
                    src_ref=src1,
                    dst_ref=comm_l.at[h + 1, s],
                    send_sem=send_l.at[h, s],
                    recv_sem=recv_l.at[h, s],
                    device_id=(left,),
                    device_id_type=pl.DeviceIdType.MESH,
                )

        rdmas = {}
        for s in range(S):
            for d in (0, 1):
                rdmas[(0, s, d)] = make(0, s, d)
                rdmas[(0, s, d)].start()

        for h in range(N_DEV - 1):
            last = h == N_DEV - 2
            for i in {0: [0], 1: [1, 2], 2: [3]}[h]:
                stage_copies[i].wait()
            for s in range(S):
                rsl = pl.ds(s * rows, rows)
                rdmas[(h, s, 0)].wait_recv()
                own_r = x_vmem[rsl, pl.ds(col_r(h), half)]
                acc_r = comm_r[h + 1, s].astype(F32) + own_r
                if not last:
                    comm_r[h + 1, s] = acc_r.astype(BF16)
                    rdmas[(h + 1, s, 0)] = make(h + 1, s, 0)
                    rdmas[(h + 1, s, 0)].start()
                else:
                    out_ref[rsl, :half] = acc_r
                rdmas[(h, s, 1)].wait_recv()
                own_l = x_vmem[rsl, pl.ds(col_l(h), half)]
                acc_l = comm_l[h + 1, s].astype(F32) + own_l
                if not last:
                    comm_l[h + 1, s] = acc_l.astype(BF16)
                    rdmas[(h + 1, s, 1)] = make(h + 1, s, 1)
                    rdmas[(h + 1, s, 1)].start()
                else:
                    out_ref[rsl, half:] = acc_l

        for r in rdmas.values():
            r.wait_send()

    return pl.pallas_call(
        body,
        out_shape=jax.ShapeDtypeStruct((m, chunk), F32),
        in_specs=[pl.BlockSpec(memory_space=pl.ANY),
                  pl.BlockSpec(memory_space=pltpu.VMEM)],
        out_specs=pl.BlockSpec(memory_space=pltpu.VMEM),
        scratch_shapes=[
            pltpu.VMEM((m, n), F32),
            pltpu.SemaphoreType.DMA((4,)),
            pltpu.VMEM((N_DEV, S, rows, half), BF16),
            pltpu.VMEM((N_DEV, S, rows, half), BF16),
            pltpu.SemaphoreType.DMA((N_DEV - 1, S)),
            pltpu.SemaphoreType.DMA((N_DEV - 1, S)),
            pltpu.SemaphoreType.DMA((N_DEV - 1, S)),
            pltpu.SemaphoreType.DMA((N_DEV - 1, S)),
        ],
        compiler_params=pltpu.CompilerParams(collective_id=0),
    )(xs, xb0)


# device time: 27528 ns/iter; 1.1287x vs baseline; 1.1287x over previous
import jax
import jax.numpy as jnp
from jax import lax
from jax.experimental import pallas as pl
from jax.experimental.pallas import tpu as pltpu

N_DEV = 4
S = 4


def kernel(x):
    _, m, n = x.shape
    xs = x.reshape(m, n)
    xb = xs.astype(jnp.bfloat16)
    chunk = n // N_DEV
    half = chunk // 2
    rows = m // S

    def body(x_hbm, out_ref, x_vmem, copy_sem, comm_r, comm_l,
             send_r, recv_r, send_l, recv_l):
        p = lax.axis_index("i")
        left = lax.rem(p + N_DEV - 1, N_DEV)
        right = lax.rem(p + 1, N_DEV)

        staging = pltpu.make_async_copy(x_hbm, x_vmem, copy_sem)
        staging.start()

        barrier_sem = pltpu.get_barrier_semaphore()
        for nbr in [left, right]:
            pl.semaphore_signal(
                barrier_sem, inc=1,
                device_id=(nbr,), device_id_type=pl.DeviceIdType.MESH,
            )
        pl.semaphore_wait(barrier_sem, 2)

        def col_r(h):
            return lax.rem(p + 2 * N_DEV - 2 - h, N_DEV) * chunk

        def col_l(h):
            return lax.rem(p + 2 + h, N_DEV) * chunk + half

        def make(h, s, direction):
            if direction == 0:
                if h == 0:
                    src = x_hbm.at[pl.ds(s * rows, rows),
                                   pl.ds(lax.rem(p + N_DEV - 1, N_DEV) * chunk,
                                         half)]
                else:
                    src = comm_r.at[h, s]
                return pltpu.make_async_remote_copy(
                    src_ref=src,
                    dst_ref=comm_r.at[h + 1, s],
                    send_sem=send_r.at[h, s],
                    recv_sem=recv_r.at[h, s],
                    device_id=(right,),
                    device_id_type=pl.DeviceIdType.MESH,
                )
            else:
                if h == 0:
                    src = x_hbm.at[pl.ds(s * rows, rows),
                                   pl.ds(lax.rem(p + 1, N_DEV) * chunk + half,
                                         half)]
                else:
                    src = comm_l.at[h, s]
                return pltpu.make_async_remote_copy(
                    src_ref=src,
                    dst_ref=comm_l.at[h + 1, s],
                    send_sem=send_l.at[h, s],
                    recv_sem=recv_l.at[h, s],
                    device_id=(left,),
                    device_id_type=pl.DeviceIdType.MESH,
                )

        rdmas = {}
        for s in range(S):
            for d in (0, 1):
                rdmas[(0, s, d)] = make(0, s, d)
                rdmas[(0, s, d)].start()

        staging.wait()

        for h in range(N_DEV - 1):
            last = h == N_DEV - 2
            for s in range(S):
                rsl = pl.ds(s * rows, rows)
                rdmas[(h, s, 0)].wait_recv()
                own_r = x_vmem[rsl, pl.ds(col_r(h), half)]
                if not last:
                    comm_r[h + 1, s] = (
                        comm_r[h + 1, s].astype(jnp.float32)
                        + own_r.astype(jnp.float32)).astype(jnp.bfloat16)
                    rdmas[(h + 1, s, 0)] = make(h + 1, s, 0)
                    rdmas[(h + 1, s, 0)].start()
                else:
                    out_ref[rsl, :half] = (
                        comm_r[h + 1, s].astype(jnp.float32)
                        + own_r.astype(jnp.float32))
                rdmas[(h, s, 1)].wait_recv()
                own_l = x_vmem[rsl, pl.ds(col_l(h), half)]
                if not last:
                    comm_l[h + 1, s] = (
                        comm_l[h + 1, s].astype(jnp.float32)
                        + own_l.astype(jnp.float32)).astype(jnp.bfloat16)
                    rdmas[(h + 1, s, 1)] = make(h + 1, s, 1)
                    rdmas[(h + 1, s, 1)].start()
                else:
                    out_ref[rsl, half:] = (
                        comm_l[h + 1, s].astype(jnp.float32)
                        + own_l.astype(jnp.float32))

        for (h, s, d), r in rdmas.items():
            r.wait_send()

    return pl.pallas_call(
        body,
        out_shape=jax.ShapeDtypeStruct((m, chunk), jnp.float32),
        in_specs=[pl.BlockSpec(memory_space=pl.ANY)],
        out_specs=pl.BlockSpec(memory_space=pltpu.VMEM),
        scratch_shapes=[
            pltpu.VMEM((m, n), jnp.bfloat16),
            pltpu.SemaphoreType.DMA,
            pltpu.VMEM((N_DEV, S, rows, half), jnp.bfloat16),
            pltpu.VMEM((N_DEV, S, rows, half), jnp.bfloat16),
            pltpu.SemaphoreType.DMA((N_DEV - 1, S)),
            pltpu.SemaphoreType.DMA((N_DEV - 1, S)),
            pltpu.SemaphoreType.DMA((N_DEV - 1, S)),
            pltpu.SemaphoreType.DMA((N_DEV - 1, S)),
        ],
        compiler_params=pltpu.CompilerParams(collective_id=0),
    )(xb)


# device time: 26266 ns/iter; 1.1829x vs baseline; 1.0480x over previous
import jax
import jax.numpy as jnp
from jax import lax
from jax.experimental import pallas as pl
from jax.experimental.pallas import tpu as pltpu

N_DEV = 4
S = 2


def kernel(x):
    _, m, n = x.shape
    xs = x.reshape(m, n)
    xb = xs.astype(jnp.bfloat16)
    chunk = n // N_DEV
    half = chunk // 2
    rows = m // S

    def body(x_hbm, out_ref, x_vmem, copy_sem, comm_r, comm_l,
             send_r, recv_r, send_l, recv_l):
        p = lax.axis_index("i")
        left = lax.rem(p + N_DEV - 1, N_DEV)
        right = lax.rem(p + 1, N_DEV)

        staging = pltpu.make_async_copy(x_hbm, x_vmem, copy_sem)
        staging.start()

        barrier_sem = pltpu.get_barrier_semaphore()
        for nbr in [left, right]:
            pl.semaphore_signal(
                barrier_sem, inc=1,
                device_id=(nbr,), device_id_type=pl.DeviceIdType.MESH,
            )
        pl.semaphore_wait(barrier_sem, 2)

        def col_r(h):
            return lax.rem(p + 2 * N_DEV - 2 - h, N_DEV) * chunk

        def col_l(h):
            return lax.rem(p + 2 + h, N_DEV) * chunk + half

        def make(h, s, direction):
            if direction == 0:
                if h == 0:
                    src = x_hbm.at[pl.ds(s * rows, rows),
                                   pl.ds(lax.rem(p + N_DEV - 1, N_DEV) * chunk,
                                         half)]
                else:
                    src = comm_r.at[h, s]
                return pltpu.make_async_remote_copy(
                    src_ref=src,
                    dst_ref=comm_r.at[h + 1, s],
                    send_sem=send_r.at[h, s],
                    recv_sem=recv_r.at[h, s],
                    device_id=(right,),
                    device_id_type=pl.DeviceIdType.MESH,
                )
            else:
                if h == 0:
                    src = x_hbm.at[pl.ds(s * rows, rows),
                                   pl.ds(lax.rem(p + 1, N_DEV) * chunk + half,
                                         half)]
                else:
                    src = comm_l.at[h, s]
                return pltpu.make_async_remote_copy(
                    src_ref=src,
                    dst_ref=comm_l.at[h + 1, s],
                    send_sem=send_l.at[h, s],
                    recv_sem=recv_l.at[h, s],
                    device_id=(left,),
                    device_id_type=pl.DeviceIdType.MESH,
                )

        rdmas = {}
        for s in range(S):
            for d in (0, 1):
                rdmas[(0, s, d)] = make(0, s, d)
                rdmas[(0, s, d)].start()

        staging.wait()

        for h in range(N_DEV - 1):
            last = h == N_DEV - 2
            for s in range(S):
                rsl = pl.ds(s * rows, rows)
                rdmas[(h, s, 0)].wait_recv()
                own_r = x_vmem[rsl, pl.ds(col_r(h), half)]
                if not last:
                    comm_r[h + 1, s] = (
                        comm_r[h + 1, s].astype(jnp.float32)
                        + own_r.astype(jnp.float32)).astype(jnp.bfloat16)
                    rdmas[(h + 1, s, 0)] = make(h + 1, s, 0)
                    rdmas[(h + 1, s, 0)].start()
                else:
                    out_ref[rsl, :half] = (
                        comm_r[h + 1, s].astype(jnp.float32)
                        + own_r.astype(jnp.float32))
                rdmas[(h, s, 1)].wait_recv()
                own_l = x_vmem[rsl, pl.ds(col_l(h), half)]
                if not last:
                    comm_l[h + 1, s] = (
                        comm_l[h + 1, s].astype(jnp.float32)
                        + own_l.astype(jnp.float32)).astype(jnp.bfloat16)
                    rdmas[(h + 1, s, 1)] = make(h + 1, s, 1)
                    rdmas[(h + 1, s, 1)].start()
                else:
                    out_ref[rsl, half:] = (
                        comm_l[h + 1, s].astype(jnp.float32)
                        + own_l.astype(jnp.float32))

        for (h, s, d), r in rdmas.items():
            r.wait_send()

    return pl.pallas_call(
        body,
        out_shape=jax.ShapeDtypeStruct((m, chunk), jnp.float32),
        in_specs=[pl.BlockSpec(memory_space=pl.ANY)],
        out_specs=pl.BlockSpec(memory_space=pltpu.VMEM),
        scratch_shapes=[
            pltpu.VMEM((m, n), jnp.bfloat16),
            pltpu.SemaphoreType.DMA,
            pltpu.VMEM((N_DEV, S, rows, half), jnp.bfloat16),
            pltpu.VMEM((N_DEV, S, rows, half), jnp.bfloat16),
            pltpu.SemaphoreType.DMA((N_DEV - 1, S)),
            pltpu.SemaphoreType.DMA((N_DEV - 1, S)),
            pltpu.SemaphoreType.DMA((N_DEV - 1, S)),
            pltpu.SemaphoreType.DMA((N_DEV - 1, S)),
        ],
        compiler_params=pltpu.CompilerParams(collective_id=0),
    )(xb)
